# Initial kernel scaffold; baseline (speedup 1.0000x reference)
import jax
import jax.numpy as jnp
from jax.experimental import pallas as pl


def kernel(x, edge_index):
    raise NotImplementedError("write your pallas kernel here")


# speedup vs baseline: 19.8763x; 19.8763x over previous
import functools

import jax
import jax.numpy as jnp
from jax import lax
from jax.experimental import pallas as pl
from jax.experimental.pallas import tpu as pltpu
from jax.experimental.pallas import tpu_sc as plsc

_NC = 2
_NS = 16
_NW = _NC * _NS
_LANES = 16
_K = 128
_C = 1.0


def _ceil_to(a, b):
    return (a + b - 1) // b * b


def _make_deg_kernel(n_pad, e_pad):
    nchunks = e_pad // (_NW * _K)
    rpt = n_pad // _NS

    mesh = plsc.VectorSubcoreMesh(core_axis_name="c", subcore_axis_name="s")

    @functools.partial(
        pl.kernel,
        out_type=jax.ShapeDtypeStruct((_NC, n_pad), jnp.float32),
        mesh=mesh,
        scratch_types=[
            pltpu.VMEM((_K,), jnp.int32),
            pltpu.VMEM((_K,), jnp.float32),
            pltpu.VMEM((rpt,), jnp.float32),
            pltpu.VMEM_SHARED((n_pad,), jnp.float32),
        ],
    )
    def deg_kernel(src_hbm, out_hbm, idx_v, ones_v, z_v, deg_sh):
        cid = lax.axis_index("c")
        sid = lax.axis_index("s")
        wid = cid * _NS + sid

        zero16 = jnp.zeros((_LANES,), jnp.float32)
        one16 = jnp.ones((_LANES,), jnp.float32)

        def zfill(i, carry):
            z_v[pl.ds(i * _LANES, _LANES)] = zero16
            return carry

        lax.fori_loop(0, rpt // _LANES, zfill, 0)
        for j in range(_K // _LANES):
            ones_v[pl.ds(j * _LANES, _LANES)] = one16

        pltpu.sync_copy(z_v, deg_sh.at[pl.ds(sid * rpt, rpt)])
        plsc.subcore_barrier()

        def body(g, carry):
            base = (wid * nchunks + g) * _K
            pltpu.sync_copy(src_hbm.at[pl.ds(base, _K)], idx_v)
            pltpu.sync_copy(ones_v, deg_sh.at[idx_v], add=True)
            return carry

        lax.fori_loop(0, nchunks, body, 0)

        plsc.subcore_barrier()
        pltpu.sync_copy(
            deg_sh.at[pl.ds(sid * rpt, rpt)],
            out_hbm.at[cid, pl.ds(sid * rpt, rpt)],
        )

    return deg_kernel


def _make_agg_kernel(n_pad, e_pad, d):
    nchunks = e_pad // (_NW * _K)
    rpt = n_pad // _NS

    mesh = plsc.VectorSubcoreMesh(core_axis_name="c", subcore_axis_name="s")

    @functools.partial(
        pl.kernel,
        out_type=jax.ShapeDtypeStruct((_NC, n_pad, d), jnp.float32),
        mesh=mesh,
        scratch_types=[
            pltpu.VMEM((_K,), jnp.int32),
            pltpu.VMEM((_K,), jnp.int32),
            pltpu.VMEM((_K, d), jnp.float32),
            pltpu.VMEM_SHARED((n_pad, d), jnp.float32),
            pltpu.SemaphoreType.DMA,
        ],
    )
    def agg_kernel(y_hbm, src_hbm, dst_hbm, out_hbm, srcv, dstv, rows_v, acc_sh, sem):
        cid = lax.axis_index("c")
        sid = lax.axis_index("s")
        wid = cid * _NS + sid

        zero16 = jnp.zeros((_LANES,), jnp.float32)

        def zfill(r, carry):
            for j in range(d // _LANES):
                rows_v[r, pl.ds(j * _LANES, _LANES)] = zero16
            return carry

        lax.fori_loop(0, _K, zfill, 0)

        for t in range(rpt // _K):
            pltpu.sync_copy(rows_v, acc_sh.at[pl.ds(sid * rpt + t * _K, _K)])
        plsc.subcore_barrier()

        def body(g, carry):
            base = (wid * nchunks + g) * _K
            pltpu.sync_copy(src_hbm.at[pl.ds(base, _K)], srcv)
            pltpu.sync_copy(dst_hbm.at[pl.ds(base, _K)], dstv)
            pltpu.async_copy(y_hbm.at[dstv], rows_v, sem).wait()
            pltpu.sync_copy(rows_v, acc_sh.at[srcv], add=True)
            return carry

        lax.fori_loop(0, nchunks, body, 0)

        plsc.subcore_barrier()
        pltpu.sync_copy(
            acc_sh.at[pl.ds(sid * rpt, rpt)],
            out_hbm.at[cid, pl.ds(sid * rpt, rpt)],
        )

    return agg_kernel


def _prep_body(dpt_ref, x_ref, y_ref, dis_ref):
    deg = dpt_ref[:, 0:1] + dpt_ref[:, 1:2]
    dis = jnp.where(deg > 0.0, lax.rsqrt(deg), 0.0)
    dis_ref[...] = dis
    y_ref[...] = x_ref[...] * dis


def _make_fin_body(n):
    def fin_body(acc_ref, dis_ref, out_ref):
        s = (acc_ref[0, :n, :] + acc_ref[1, :n, :]) * dis_ref[...]
        sq = jnp.sum(s * s, axis=1, keepdims=True)
        t0 = s[:, 0:1]
        li = sq - 2.0 * t0 * t0
        denom = jnp.sqrt(jnp.clip(jnp.abs(li), 1e-8, None))
        out_ref[...] = s * ((1.0 / jnp.sqrt(_C)) / denom)

    return fin_body


def kernel(x, edge_index):
    n, d = x.shape
    e = edge_index.shape[1]
    n_pad = _ceil_to(n + 1, _NS * _K)
    e_pad = _ceil_to(e, _NW * _K)
    pad = e_pad - e

    src = edge_index[0]
    dst = edge_index[1]
    if pad:
        ar = jnp.arange(pad, dtype=jnp.int32)
        src = jnp.concatenate([src, n + ar % (n_pad - n)])
        dst = jnp.concatenate([dst, ar % n])

    dp = _make_deg_kernel(n_pad, e_pad)(src)
    dpt = jnp.transpose(dp)[:n]

    y, dis = pl.pallas_call(
        _prep_body,
        out_shape=(
            jax.ShapeDtypeStruct((n, d), jnp.float32),
            jax.ShapeDtypeStruct((n, 1), jnp.float32),
        ),
    )(dpt, x)

    acc = _make_agg_kernel(n_pad, e_pad, d)(y, src, dst)

    out = pl.pallas_call(
        _make_fin_body(n),
        out_shape=jax.ShapeDtypeStruct((n, d), jnp.float32),
    )(acc, dis)
    return out


# speedup vs baseline: 30.7409x; 1.5466x over previous
import functools

import jax
import jax.numpy as jnp
from jax import lax
from jax.experimental import pallas as pl
from jax.experimental.pallas import tpu as pltpu
from jax.experimental.pallas import tpu_sc as plsc

_NC = 2
_NS = 16
_NW = _NC * _NS
_LANES = 16
_K = 128
_NB = 2
_NBI = 2 * _NB
_C = 1.0


def _ceil_to(a, b):
    return (a + b - 1) // b * b


def _make_deg_kernel(n_pad, e_pad):
    nchunks = e_pad // (_NW * _K)
    rpt = n_pad // _NS

    mesh = plsc.VectorSubcoreMesh(core_axis_name="c", subcore_axis_name="s")

    @functools.partial(
        pl.kernel,
        out_type=jax.ShapeDtypeStruct((_NC, n_pad), jnp.float32),
        mesh=mesh,
        scratch_types=[
            pltpu.VMEM((2, _K), jnp.int32),
            pltpu.VMEM((_K,), jnp.float32),
            pltpu.VMEM((rpt,), jnp.float32),
            pltpu.VMEM_SHARED((n_pad,), jnp.float32),
            pltpu.SemaphoreType.DMA((2,)),
        ],
    )
    def deg_kernel(src_hbm, out_hbm, idx_v, ones_v, z_v, deg_sh, sem_i):
        cid = lax.axis_index("c")
        sid = lax.axis_index("s")
        wid = cid * _NS + sid
        cbase = wid * nchunks

        zero16 = jnp.zeros((_LANES,), jnp.float32)
        one16 = jnp.ones((_LANES,), jnp.float32)

        def zfill(i, carry):
            z_v[pl.ds(i * _LANES, _LANES)] = zero16
            return carry

        lax.fori_loop(0, rpt // _LANES, zfill, 0)
        for j in range(_K // _LANES):
            ones_v[pl.ds(j * _LANES, _LANES)] = one16

        pltpu.sync_copy(z_v, deg_sh.at[pl.ds(sid * rpt, rpt)])
        plsc.subcore_barrier()

        def issue_idx(i, b):
            pltpu.async_copy(
                src_hbm.at[pl.ds((cbase + i) * _K, _K)], idx_v.at[b], sem_i.at[b]
            )

        def wait_idx(i, b):
            pltpu.make_async_copy(
                src_hbm.at[pl.ds((cbase + i) * _K, _K)], idx_v.at[b], sem_i.at[b]
            ).wait()

        issue_idx(0, 0)

        def pair(t, carry):
            for b in range(2):
                i = 2 * t + b
                wait_idx(i, b)
                issue_idx(i + 1, 1 - b)
                pltpu.sync_copy(ones_v, deg_sh.at[idx_v.at[b]], add=True)
            return carry

        lax.fori_loop(0, nchunks // 2 - 1, pair, 0)
        i = nchunks - 2
        wait_idx(i, 0)
        issue_idx(i + 1, 1)
        pltpu.sync_copy(ones_v, deg_sh.at[idx_v.at[0]], add=True)
        wait_idx(i + 1, 1)
        pltpu.sync_copy(ones_v, deg_sh.at[idx_v.at[1]], add=True)

        plsc.subcore_barrier()
        pltpu.sync_copy(
            deg_sh.at[pl.ds(sid * rpt, rpt)],
            out_hbm.at[cid, pl.ds(sid * rpt, rpt)],
        )

    return deg_kernel


def _make_agg_kernel(n_pad, e_pad, d):
    nchunks = e_pad // (_NW * _K)
    rpt = n_pad // _NS
    nsg = nchunks // _NBI

    mesh = plsc.VectorSubcoreMesh(core_axis_name="c", subcore_axis_name="s")

    @functools.partial(
        pl.kernel,
        out_type=jax.ShapeDtypeStruct((_NC, n_pad, d), jnp.float32),
        mesh=mesh,
        scratch_types=[
            pltpu.VMEM((_NBI, _K), jnp.int32),
            pltpu.VMEM((_NBI, _K), jnp.int32),
            pltpu.VMEM((_NB, _K, d), jnp.float32),
            pltpu.VMEM_SHARED((n_pad, d), jnp.float32),
            pltpu.SemaphoreType.DMA((_NBI,)),
            pltpu.SemaphoreType.DMA((_NB,)),
            pltpu.SemaphoreType.DMA((_NB,)),
        ],
    )
    def agg_kernel(
        y_hbm, src_hbm, dst_hbm, out_hbm,
        srcv, dstv, rows, acc_sh, sem_i, sem_g, sem_s,
    ):
        cid = lax.axis_index("c")
        sid = lax.axis_index("s")
        wid = cid * _NS + sid
        cbase = wid * nchunks

        zero16 = jnp.zeros((_LANES,), jnp.float32)

        def zfill(r, carry):
            for j in range(d // _LANES):
                rows[0, r, pl.ds(j * _LANES, _LANES)] = zero16
            return carry

        lax.fori_loop(0, _K, zfill, 0)
        for t in range(rpt // _K):
            pltpu.sync_copy(rows.at[0], acc_sh.at[pl.ds(sid * rpt + t * _K, _K)])
        plsc.subcore_barrier()

        def issue_idx(i, bI):
            pltpu.async_copy(
                src_hbm.at[pl.ds((cbase + i) * _K, _K)], srcv.at[bI], sem_i.at[bI]
            )
            pltpu.async_copy(
                dst_hbm.at[pl.ds((cbase + i) * _K, _K)], dstv.at[bI], sem_i.at[bI]
            )

        def wait_idx(i, bI):
            pltpu.make_async_copy(
                src_hbm.at[pl.ds((cbase + i) * _K, _K)], srcv.at[bI], sem_i.at[bI]
            ).wait()
            pltpu.make_async_copy(
                dst_hbm.at[pl.ds((cbase + i) * _K, _K)], dstv.at[bI], sem_i.at[bI]
            ).wait()

        def issue_gather(bI, bR):
            pltpu.async_copy(y_hbm.at[dstv.at[bI]], rows.at[bR], sem_g.at[bR])

        def wait_gather(bI, bR):
            pltpu.make_async_copy(
                y_hbm.at[dstv.at[bI]], rows.at[bR], sem_g.at[bR]
            ).wait()

        def issue_scat(bI, bR):
            pltpu.async_copy(
                rows.at[bR], acc_sh.at[srcv.at[bI]], sem_s.at[bR], add=True
            )

        def wait_scat(bI, bR):
            pltpu.make_async_copy(
                rows.at[bR], acc_sh.at[srcv.at[bI]], sem_s.at[bR]
            ).wait()

        def do_chunk(i, k, wait_prev_scat, idx_ahead, gather_next):
            wait_gather(k, k % _NB)
            issue_scat(k, k % _NB)
            if wait_prev_scat:
                wait_scat((k - 1) % _NBI, (k - 1) % _NB)
            if idx_ahead:
                issue_idx(i + 2, (k + 2) % _NBI)
            if gather_next:
                wait_idx(i + 1, (k + 1) % _NBI)
                issue_gather((k + 1) % _NBI, (k + 1) % _NB)

        issue_idx(0, 0)
        issue_idx(1, 1)
        wait_idx(0, 0)
        issue_gather(0, 0)

        do_chunk(0, 0, False, True, True)
        for k in range(1, _NBI):
            do_chunk(k, k, True, True, True)

        def sgroup(u, carry):
            i0 = u * _NBI
            for k in range(_NBI):
                do_chunk(i0 + k, k, True, True, True)
            return carry

        lax.fori_loop(1, nsg - 1, sgroup, 0)

        i0 = (nsg - 1) * _NBI
        for k in range(_NBI):
            i = i0 + k
            do_chunk(i, k, True, i + 2 < nchunks, i + 1 < nchunks)

        k = (nchunks - 1) % _NBI
        wait_scat(k, k % _NB)

        plsc.subcore_barrier()
        pltpu.sync_copy(
            acc_sh.at[pl.ds(sid * rpt, rpt)],
            out_hbm.at[cid, pl.ds(sid * rpt, rpt)],
        )

    return agg_kernel


def _prep_body(dpt_ref, x_ref, y_ref, dis_ref):
    deg = dpt_ref[:, 0:1] + dpt_ref[:, 1:2]
    dis = jnp.where(deg > 0.0, lax.rsqrt(deg), 0.0)
    dis_ref[...] = dis
    y_ref[...] = x_ref[...] * dis


def _make_fin_body(n):
    def fin_body(acc_ref, dis_ref, out_ref):
        s = (acc_ref[0, :n, :] + acc_ref[1, :n, :]) * dis_ref[...]
        sq = jnp.sum(s * s, axis=1, keepdims=True)
        t0 = s[:, 0:1]
        li = sq - 2.0 * t0 * t0
        denom = jnp.sqrt(jnp.clip(jnp.abs(li), 1e-8, None))
        out_ref[...] = s * ((1.0 / jnp.sqrt(_C)) / denom)

    return fin_body


def kernel(x, edge_index):
    n, d = x.shape
    e = edge_index.shape[1]
    n_pad = _ceil_to(n + 1, _NS * _K)
    e_pad = _ceil_to(e, _NW * _K * _NBI)
    pad = e_pad - e

    src = edge_index[0]
    dst = edge_index[1]
    if pad:
        ar = jnp.arange(pad, dtype=jnp.int32)
        src = jnp.concatenate([src, n + ar % (n_pad - n)])
        dst = jnp.concatenate([dst, ar % n])

    dp = _make_deg_kernel(n_pad, e_pad)(src)
    dpt = jnp.transpose(dp)[:n]

    y, dis = pl.pallas_call(
        _prep_body,
        out_shape=(
            jax.ShapeDtypeStruct((n, d), jnp.float32),
            jax.ShapeDtypeStruct((n, 1), jnp.float32),
        ),
    )(dpt, x)

    acc = _make_agg_kernel(n_pad, e_pad, d)(y, src, dst)

    out = pl.pallas_call(
        _make_fin_body(n),
        out_shape=jax.ShapeDtypeStruct((n, d), jnp.float32),
    )(acc, dis)
    return out


# speedup vs baseline: 30.7542x; 1.0004x over previous
import functools

import jax
import jax.numpy as jnp
from jax import lax
from jax.experimental import pallas as pl
from jax.experimental.pallas import tpu as pltpu
from jax.experimental.pallas import tpu_sc as plsc

_NC = 2
_NS = 16
_NW = _NC * _NS
_LANES = 16
_K = 128
_NB = 2
_NBI = 2 * _NB
_C = 1.0


def _ceil_to(a, b):
    return (a + b - 1) // b * b


def _make_deg_kernel(n_pad, e_pad):
    nchunks = e_pad // (_NW * _K)
    rpt = n_pad // _NS

    mesh = plsc.VectorSubcoreMesh(core_axis_name="c", subcore_axis_name="s")

    @functools.partial(
        pl.kernel,
        out_type=(
            jax.ShapeDtypeStruct((n_pad,), jnp.float32),
            jax.ShapeDtypeStruct((n_pad,), jnp.float32),
        ),
        mesh=mesh,
        scratch_types=[
            pltpu.VMEM((2, _K), jnp.int32),
            pltpu.VMEM((_K,), jnp.float32),
            pltpu.VMEM((rpt,), jnp.float32),
            pltpu.VMEM_SHARED((n_pad,), jnp.float32),
            pltpu.SemaphoreType.DMA((2,)),
        ],
    )
    def deg_kernel(edge_hbm, out0_hbm, out1_hbm, idx_v, ones_v, z_v, deg_sh, sem_i):
        cid = lax.axis_index("c")
        sid = lax.axis_index("s")
        wid = cid * _NS + sid
        cbase = wid * nchunks

        zero16 = jnp.zeros((_LANES,), jnp.float32)
        one16 = jnp.ones((_LANES,), jnp.float32)

        def zfill(i, carry):
            z_v[pl.ds(i * _LANES, _LANES)] = zero16
            return carry

        lax.fori_loop(0, rpt // _LANES, zfill, 0)
        for j in range(_K // _LANES):
            ones_v[pl.ds(j * _LANES, _LANES)] = one16

        pltpu.sync_copy(z_v, deg_sh.at[pl.ds(sid * rpt, rpt)])
        plsc.subcore_barrier()

        def issue_idx(i, b):
            pltpu.async_copy(
                edge_hbm.at[0, pl.ds((cbase + i) * _K, _K)], idx_v.at[b], sem_i.at[b]
            )

        def wait_idx(i, b):
            pltpu.make_async_copy(
                edge_hbm.at[0, pl.ds((cbase + i) * _K, _K)], idx_v.at[b], sem_i.at[b]
            ).wait()

        issue_idx(0, 0)

        def pair(t, carry):
            for b in range(2):
                i = 2 * t + b
                wait_idx(i, b)
                issue_idx(i + 1, 1 - b)
                pltpu.sync_copy(ones_v, deg_sh.at[idx_v.at[b]], add=True)
            return carry

        lax.fori_loop(0, nchunks // 2 - 1, pair, 0)
        i = nchunks - 2
        wait_idx(i, 0)
        issue_idx(i + 1, 1)
        pltpu.sync_copy(ones_v, deg_sh.at[idx_v.at[0]], add=True)
        wait_idx(i + 1, 1)
        pltpu.sync_copy(ones_v, deg_sh.at[idx_v.at[1]], add=True)

        plsc.subcore_barrier()

        @pl.when(cid == 0)
        def _():
            pltpu.sync_copy(
                deg_sh.at[pl.ds(sid * rpt, rpt)],
                out0_hbm.at[pl.ds(sid * rpt, rpt)],
            )

        @pl.when(cid == 1)
        def _():
            pltpu.sync_copy(
                deg_sh.at[pl.ds(sid * rpt, rpt)],
                out1_hbm.at[pl.ds(sid * rpt, rpt)],
            )

    return deg_kernel


def _make_agg_kernel(n_pad, e_pad, d):
    nchunks = e_pad // (_NW * _K)
    rpt = n_pad // _NS
    nsg = nchunks // _NBI

    mesh = plsc.VectorSubcoreMesh(core_axis_name="c", subcore_axis_name="s")

    @functools.partial(
        pl.kernel,
        out_type=jax.ShapeDtypeStruct((_NC, n_pad, d), jnp.float32),
        mesh=mesh,
        scratch_types=[
            pltpu.VMEM((_NBI, _K), jnp.int32),
            pltpu.VMEM((_NBI, _K), jnp.int32),
            pltpu.VMEM((_NB, _K, d), jnp.float32),
            pltpu.VMEM_SHARED((n_pad, d), jnp.float32),
            pltpu.SemaphoreType.DMA((_NBI,)),
            pltpu.SemaphoreType.DMA((_NB,)),
            pltpu.SemaphoreType.DMA((_NB,)),
        ],
    )
    def agg_kernel(
        y_hbm, edge_hbm, out_hbm,
        srcv, dstv, rows, acc_sh, sem_i, sem_g, sem_s,
    ):
        cid = lax.axis_index("c")
        sid = lax.axis_index("s")
        wid = cid * _NS + sid
        cbase = wid * nchunks

        zero16 = jnp.zeros((_LANES,), jnp.float32)

        def zfill(r, carry):
            for j in range(d // _LANES):
                rows[0, r, pl.ds(j * _LANES, _LANES)] = zero16
            return carry

        lax.fori_loop(0, _K, zfill, 0)
        for t in range(rpt // _K):
            pltpu.sync_copy(rows.at[0], acc_sh.at[pl.ds(sid * rpt + t * _K, _K)])
        plsc.subcore_barrier()

        def issue_idx(i, bI):
            pltpu.async_copy(
                edge_hbm.at[0, pl.ds((cbase + i) * _K, _K)], srcv.at[bI], sem_i.at[bI]
            )
            pltpu.async_copy(
                edge_hbm.at[1, pl.ds((cbase + i) * _K, _K)], dstv.at[bI], sem_i.at[bI]
            )

        def wait_idx(i, bI):
            pltpu.make_async_copy(
                edge_hbm.at[0, pl.ds((cbase + i) * _K, _K)], srcv.at[bI], sem_i.at[bI]
            ).wait()
            pltpu.make_async_copy(
                edge_hbm.at[1, pl.ds((cbase + i) * _K, _K)], dstv.at[bI], sem_i.at[bI]
            ).wait()

        def issue_gather(bI, bR):
            pltpu.async_copy(y_hbm.at[dstv.at[bI]], rows.at[bR], sem_g.at[bR])

        def wait_gather(bI, bR):
            pltpu.make_async_copy(
                y_hbm.at[dstv.at[bI]], rows.at[bR], sem_g.at[bR]
            ).wait()

        def issue_scat(bI, bR):
            pltpu.async_copy(
                rows.at[bR], acc_sh.at[srcv.at[bI]], sem_s.at[bR], add=True
            )

        def wait_scat(bI, bR):
            pltpu.make_async_copy(
                rows.at[bR], acc_sh.at[srcv.at[bI]], sem_s.at[bR]
            ).wait()

        def do_chunk(i, k, wait_prev_scat, idx_ahead, gather_next):
            wait_gather(k, k % _NB)
            issue_scat(k, k % _NB)
            if wait_prev_scat:
                wait_scat((k - 1) % _NBI, (k - 1) % _NB)
            if idx_ahead:
                issue_idx(i + 2, (k + 2) % _NBI)
            if gather_next:
                wait_idx(i + 1, (k + 1) % _NBI)
                issue_gather((k + 1) % _NBI, (k + 1) % _NB)

        issue_idx(0, 0)
        issue_idx(1, 1)
        wait_idx(0, 0)
        issue_gather(0, 0)

        do_chunk(0, 0, False, True, True)
        for k in range(1, _NBI):
            do_chunk(k, k, True, True, True)

        def sgroup(u, carry):
            i0 = u * _NBI
            for k in range(_NBI):
                do_chunk(i0 + k, k, True, True, True)
            return carry

        lax.fori_loop(1, nsg - 1, sgroup, 0)

        i0 = (nsg - 1) * _NBI
        for k in range(_NBI):
            i = i0 + k
            do_chunk(i, k, True, i + 2 < nchunks, i + 1 < nchunks)

        k = (nchunks - 1) % _NBI
        wait_scat(k, k % _NB)

        plsc.subcore_barrier()
        pltpu.sync_copy(
            acc_sh.at[pl.ds(sid * rpt, rpt)],
            out_hbm.at[cid, pl.ds(sid * rpt, rpt)],
        )

    return agg_kernel


def _prep_body(d0_ref, d1_ref, x_ref, y_ref):
    deg = d0_ref[...] + d1_ref[...]
    dis = jnp.where(deg > 0.0, lax.rsqrt(deg), 0.0)
    y_ref[...] = x_ref[...] * dis


def _make_fin_body(n):
    def fin_body(acc_ref, d0_ref, d1_ref, out_ref):
        deg = d0_ref[...] + d1_ref[...]
        dis = jnp.where(deg > 0.0, lax.rsqrt(deg), 0.0)
        s = (acc_ref[0, :n, :] + acc_ref[1, :n, :]) * dis
        sq = jnp.sum(s * s, axis=1, keepdims=True)
        t0 = s[:, 0:1]
        li = sq - 2.0 * t0 * t0
        denom = jnp.sqrt(jnp.clip(jnp.abs(li), 1e-8, None))
        out_ref[...] = s * ((1.0 / jnp.sqrt(_C)) / denom)

    return fin_body


def kernel(x, edge_index):
    n, d = x.shape
    e = edge_index.shape[1]
    n_pad = _ceil_to(n + 1, _NS * _K)
    e_pad = _ceil_to(e, _NW * _K * _NBI)
    pad = e_pad - e

    edge_p = edge_index
    if pad:
        ar = jnp.arange(pad, dtype=jnp.int32)
        pad_blk = jnp.stack([n + ar % (n_pad - n), ar % n])
        edge_p = jnp.concatenate([edge_index, pad_blk], axis=1)

    dp0, dp1 = _make_deg_kernel(n_pad, e_pad)(edge_p)
    d0 = dp0[:n, None]
    d1 = dp1[:n, None]

    y = pl.pallas_call(
        _prep_body,
        out_shape=jax.ShapeDtypeStruct((n, d), jnp.float32),
    )(d0, d1, x)

    acc = _make_agg_kernel(n_pad, e_pad, d)(y, edge_p)

    out = pl.pallas_call(
        _make_fin_body(n),
        out_shape=jax.ShapeDtypeStruct((n, d), jnp.float32),
    )(acc, d0, d1)
    return out


# speedup vs baseline: 31.3545x; 1.0195x over previous
import functools

import jax
import jax.numpy as jnp
from jax import lax
from jax.experimental import pallas as pl
from jax.experimental.pallas import tpu as pltpu
from jax.experimental.pallas import tpu_sc as plsc

_NC = 2
_NS = 16
_NW = _NC * _NS
_LANES = 16
_K = 128
_NB = 2
_NBI = 2 * _NB
_C = 1.0


def _ceil_to(a, b):
    return (a + b - 1) // b * b


def _make_deg_kernel(n_pad, e):
    per_tile = e // _NW
    nchunks = per_tile // _K
    tail = per_tile % _K
    assert nchunks >= 2 and nchunks % 2 == 0 and tail % 8 == 0
    rpt = n_pad // _NS

    mesh = plsc.VectorSubcoreMesh(core_axis_name="c", subcore_axis_name="s")

    @functools.partial(
        pl.kernel,
        out_type=(
            jax.ShapeDtypeStruct((n_pad,), jnp.float32),
            jax.ShapeDtypeStruct((n_pad,), jnp.float32),
        ),
        mesh=mesh,
        scratch_types=[
            pltpu.VMEM((2, _K), jnp.int32),
            pltpu.VMEM((tail if tail else 8,), jnp.int32),
            pltpu.VMEM((_K,), jnp.float32),
            pltpu.VMEM((rpt,), jnp.float32),
            pltpu.VMEM_SHARED((n_pad,), jnp.float32),
            pltpu.SemaphoreType.DMA((2,)),
        ],
    )
    def deg_kernel(edge_hbm, out0_hbm, out1_hbm, idx_v, idx_t, ones_v, z_v, deg_sh, sem_i):
        cid = lax.axis_index("c")
        sid = lax.axis_index("s")
        wid = cid * _NS + sid
        ebase = wid * per_tile

        zero16 = jnp.zeros((_LANES,), jnp.float32)
        one16 = jnp.ones((_LANES,), jnp.float32)

        def zfill(i, carry):
            z_v[pl.ds(i * _LANES, _LANES)] = zero16
            return carry

        lax.fori_loop(0, rpt // _LANES, zfill, 0)
        for j in range(_K // _LANES):
            ones_v[pl.ds(j * _LANES, _LANES)] = one16

        pltpu.sync_copy(z_v, deg_sh.at[pl.ds(sid * rpt, rpt)])
        plsc.subcore_barrier()

        def issue_idx(i, b):
            pltpu.async_copy(
                edge_hbm.at[pl.ds(ebase + i * _K, _K)], idx_v.at[b], sem_i.at[b]
            )

        def wait_idx(i, b):
            pltpu.make_async_copy(
                edge_hbm.at[pl.ds(ebase + i * _K, _K)], idx_v.at[b], sem_i.at[b]
            ).wait()

        issue_idx(0, 0)

        def pair(t, carry):
            for b in range(2):
                i = 2 * t + b
                wait_idx(i, b)
                issue_idx(i + 1, 1 - b)
                pltpu.sync_copy(ones_v, deg_sh.at[idx_v.at[b]], add=True)
            return carry

        lax.fori_loop(0, nchunks // 2 - 1, pair, 0)
        i = nchunks - 2
        wait_idx(i, 0)
        issue_idx(i + 1, 1)
        pltpu.sync_copy(ones_v, deg_sh.at[idx_v.at[0]], add=True)
        wait_idx(i + 1, 1)
        pltpu.sync_copy(ones_v, deg_sh.at[idx_v.at[1]], add=True)
        if tail:
            pltpu.sync_copy(
                edge_hbm.at[pl.ds(ebase + nchunks * _K, tail)], idx_t
            )
            pltpu.sync_copy(
                ones_v.at[pl.ds(0, tail)], deg_sh.at[idx_t], add=True
            )

        plsc.subcore_barrier()

        @pl.when(cid == 0)
        def _():
            pltpu.sync_copy(
                deg_sh.at[pl.ds(sid * rpt, rpt)],
                out0_hbm.at[pl.ds(sid * rpt, rpt)],
            )

        @pl.when(cid == 1)
        def _():
            pltpu.sync_copy(
                deg_sh.at[pl.ds(sid * rpt, rpt)],
                out1_hbm.at[pl.ds(sid * rpt, rpt)],
            )

    return deg_kernel


def _make_agg_kernel(n_pad, e, d):
    per_tile = e // _NW
    nchunks = per_tile // _K
    tail = per_tile % _K
    rpt = n_pad // _NS
    nsg = nchunks // _NBI
    rem = nchunks % _NBI
    assert nsg >= 2 and tail % 8 == 0

    mesh = plsc.VectorSubcoreMesh(core_axis_name="c", subcore_axis_name="s")

    @functools.partial(
        pl.kernel,
        out_type=jax.ShapeDtypeStruct((_NC, n_pad, d), jnp.float32),
        mesh=mesh,
        scratch_types=[
            pltpu.VMEM((_NBI, _K), jnp.int32),
            pltpu.VMEM((_NBI, _K), jnp.int32),
            pltpu.VMEM((tail if tail else 8,), jnp.int32),
            pltpu.VMEM((tail if tail else 8,), jnp.int32),
            pltpu.VMEM((_NB, _K, d), jnp.float32),
            pltpu.VMEM_SHARED((n_pad, d), jnp.float32),
            pltpu.SemaphoreType.DMA((_NBI,)),
            pltpu.SemaphoreType.DMA((_NB,)),
            pltpu.SemaphoreType.DMA((_NB,)),
        ],
    )
    def agg_kernel(
        y_hbm, edge_hbm, out_hbm,
        srcv, dstv, srcv_t, dstv_t, rows, acc_sh, sem_i, sem_g, sem_s,
    ):
        cid = lax.axis_index("c")
        sid = lax.axis_index("s")
        wid = cid * _NS + sid
        ebase = wid * per_tile

        zero16 = jnp.zeros((_LANES,), jnp.float32)

        def zfill(r, carry):
            for j in range(d // _LANES):
                rows[0, r, pl.ds(j * _LANES, _LANES)] = zero16
            return carry

        lax.fori_loop(0, _K, zfill, 0)
        for t in range(rpt // _K):
            pltpu.sync_copy(rows.at[0], acc_sh.at[pl.ds(sid * rpt + t * _K, _K)])
        plsc.subcore_barrier()

        def issue_idx(i, bI):
            pltpu.async_copy(
                edge_hbm.at[pl.ds(ebase + i * _K, _K)], srcv.at[bI], sem_i.at[bI]
            )
            pltpu.async_copy(
                edge_hbm.at[pl.ds(e + ebase + i * _K, _K)], dstv.at[bI], sem_i.at[bI]
            )

        def wait_idx(i, bI):
            pltpu.make_async_copy(
                edge_hbm.at[pl.ds(ebase + i * _K, _K)], srcv.at[bI], sem_i.at[bI]
            ).wait()
            pltpu.make_async_copy(
                edge_hbm.at[pl.ds(e + ebase + i * _K, _K)], dstv.at[bI], sem_i.at[bI]
            ).wait()

        def issue_gather(bI, bR):
            pltpu.async_copy(y_hbm.at[dstv.at[bI]], rows.at[bR], sem_g.at[bR])

        def wait_gather(bI, bR):
            pltpu.make_async_copy(
                y_hbm.at[dstv.at[bI]], rows.at[bR], sem_g.at[bR]
            ).wait()

        def issue_scat(bI, bR):
            pltpu.async_copy(
                rows.at[bR], acc_sh.at[srcv.at[bI]], sem_s.at[bR], add=True
            )

        def wait_scat(bI, bR):
            pltpu.make_async_copy(
                rows.at[bR], acc_sh.at[srcv.at[bI]], sem_s.at[bR]
            ).wait()

        def do_chunk(i, k, wait_prev_scat, idx_ahead, gather_next):
            wait_gather(k, k % _NB)
            issue_scat(k, k % _NB)
            if wait_prev_scat:
                wait_scat((k - 1) % _NBI, (k - 1) % _NB)
            if idx_ahead:
                issue_idx(i + 2, (k + 2) % _NBI)
            if gather_next:
                wait_idx(i + 1, (k + 1) % _NBI)
                issue_gather((k + 1) % _NBI, (k + 1) % _NB)

        issue_idx(0, 0)
        issue_idx(1, 1)
        wait_idx(0, 0)
        issue_gather(0, 0)

        do_chunk(0, 0, False, True, True)
        for k in range(1, _NBI):
            do_chunk(k, k, True, True, True)

        def sgroup(u, carry):
            i0 = u * _NBI
            for k in range(_NBI):
                do_chunk(i0 + k, k, True, True, True)
            return carry

        if rem == 0:
            lax.fori_loop(1, nsg - 1, sgroup, 0)
            i0 = (nsg - 1) * _NBI
            for k in range(_NBI):
                i = i0 + k
                do_chunk(i, k, True, i + 2 < nchunks, i + 1 < nchunks)
        else:
            lax.fori_loop(1, nsg, sgroup, 0)
            i0 = nsg * _NBI
            for k in range(rem):
                i = i0 + k
                do_chunk(i, k, True, i + 2 < nchunks, i + 1 < nchunks)

        kl = (nchunks - 1) % _NBI
        wait_scat(kl, kl % _NB)

        if tail:
            tb = ebase + nchunks * _K
            pltpu.sync_copy(edge_hbm.at[pl.ds(tb, tail)], srcv_t)
            pltpu.sync_copy(edge_hbm.at[pl.ds(e + tb, tail)], dstv_t)
            pltpu.async_copy(
                y_hbm.at[dstv_t], rows.at[0, pl.ds(0, tail)], sem_g.at[0]
            ).wait()
            pltpu.sync_copy(
                rows.at[0, pl.ds(0, tail)], acc_sh.at[srcv_t], add=True
            )

        plsc.subcore_barrier()
        pltpu.sync_copy(
            acc_sh.at[pl.ds(sid * rpt, rpt)],
            out_hbm.at[cid, pl.ds(sid * rpt, rpt)],
        )

    return agg_kernel


def _prep_body(d0_ref, d1_ref, x_ref, y_ref):
    deg = d0_ref[...] + d1_ref[...]
    dis = jnp.where(deg > 0.0, lax.rsqrt(deg), 0.0)
    y_ref[...] = x_ref[...] * dis


def _make_fin_body(n):
    def fin_body(acc_ref, d0_ref, d1_ref, out_ref):
        deg = d0_ref[...] + d1_ref[...]
        dis = jnp.where(deg > 0.0, lax.rsqrt(deg), 0.0)
        s = (acc_ref[0, :n, :] + acc_ref[1, :n, :]) * dis
        sq = jnp.sum(s * s, axis=1, keepdims=True)
        t0 = s[:, 0:1]
        li = sq - 2.0 * t0 * t0
        denom = jnp.sqrt(jnp.clip(jnp.abs(li), 1e-8, None))
        out_ref[...] = s * ((1.0 / jnp.sqrt(_C)) / denom)

    return fin_body


def kernel(x, edge_index):
    n, d = x.shape
    e = edge_index.shape[1]
    n_pad = _ceil_to(n, _NS * _K)

    edge_flat = edge_index.reshape(-1)

    dp0, dp1 = _make_deg_kernel(n_pad, e)(edge_flat)
    d0 = dp0[:n, None]
    d1 = dp1[:n, None]

    y = pl.pallas_call(
        _prep_body,
        out_shape=jax.ShapeDtypeStruct((n, d), jnp.float32),
    )(d0, d1, x)

    acc = _make_agg_kernel(n_pad, e, d)(y, edge_flat)

    out = pl.pallas_call(
        _make_fin_body(n),
        out_shape=jax.ShapeDtypeStruct((n, d), jnp.float32),
    )(acc, d0, d1)
    return out


# speedup vs baseline: 32.9334x; 1.0504x over previous
import functools

import jax
import jax.numpy as jnp
from jax import lax
from jax.experimental import pallas as pl
from jax.experimental.pallas import tpu as pltpu
from jax.experimental.pallas import tpu_sc as plsc

_NC = 2
_NS = 16
_NW = _NC * _NS
_LANES = 16
_K = 128
_NB = 2
_NBI = 2 * _NB
_C = 1.0


def _ceil_to(a, b):
    return (a + b - 1) // b * b


def _make_deg_kernel(n_pad, e):
    per_tile = e // _NW
    nchunks = per_tile // _K
    tail = per_tile % _K
    rpt = n_pad // _NS
    assert nchunks >= 2 and nchunks % 2 == 0
    assert tail % 8 == 0 and rpt % _LANES == 0

    mesh = plsc.VectorSubcoreMesh(core_axis_name="c", subcore_axis_name="s")

    @functools.partial(
        pl.kernel,
        out_type=(
            jax.ShapeDtypeStruct((n_pad,), jnp.float32),
            jax.ShapeDtypeStruct((n_pad,), jnp.float32),
        ),
        mesh=mesh,
        scratch_types=[
            pltpu.VMEM((2, _K), jnp.int32),
            pltpu.VMEM((tail if tail else 8,), jnp.int32),
            pltpu.VMEM((_K,), jnp.float32),
            pltpu.VMEM((rpt,), jnp.float32),
            pltpu.VMEM_SHARED((n_pad,), jnp.float32),
            pltpu.SemaphoreType.DMA((2,)),
        ],
    )
    def deg_kernel(edge_hbm, out0_hbm, out1_hbm, idx_v, idx_t, ones_v, z_v, deg_sh, sem_i):
        cid = lax.axis_index("c")
        sid = lax.axis_index("s")
        wid = cid * _NS + sid
        ebase = wid * per_tile

        zero16 = jnp.zeros((_LANES,), jnp.float32)
        one16 = jnp.ones((_LANES,), jnp.float32)

        def zfill(i, carry):
            z_v[pl.ds(i * _LANES, _LANES)] = zero16
            return carry

        lax.fori_loop(0, rpt // _LANES, zfill, 0)
        for j in range(_K // _LANES):
            ones_v[pl.ds(j * _LANES, _LANES)] = one16

        pltpu.sync_copy(z_v, deg_sh.at[pl.ds(sid * rpt, rpt)])
        plsc.subcore_barrier()

        def issue_idx(i, b):
            pltpu.async_copy(
                edge_hbm.at[pl.ds(ebase + i * _K, _K)], idx_v.at[b], sem_i.at[b]
            )

        def wait_idx(i, b):
            pltpu.make_async_copy(
                edge_hbm.at[pl.ds(ebase + i * _K, _K)], idx_v.at[b], sem_i.at[b]
            ).wait()

        issue_idx(0, 0)

        def pair(t, carry):
            for b in range(2):
                i = 2 * t + b
                wait_idx(i, b)
                issue_idx(i + 1, 1 - b)
                pltpu.sync_copy(ones_v, deg_sh.at[idx_v.at[b]], add=True)
            return carry

        lax.fori_loop(0, nchunks // 2 - 1, pair, 0)
        i = nchunks - 2
        wait_idx(i, 0)
        issue_idx(i + 1, 1)
        pltpu.sync_copy(ones_v, deg_sh.at[idx_v.at[0]], add=True)
        wait_idx(i + 1, 1)
        pltpu.sync_copy(ones_v, deg_sh.at[idx_v.at[1]], add=True)
        if tail:
            pltpu.sync_copy(edge_hbm.at[pl.ds(ebase + nchunks * _K, tail)], idx_t)
            pltpu.sync_copy(ones_v.at[pl.ds(0, tail)], deg_sh.at[idx_t], add=True)

        plsc.subcore_barrier()

        @pl.when(cid == 0)
        def _():
            pltpu.sync_copy(
                deg_sh.at[pl.ds(sid * rpt, rpt)],
                out0_hbm.at[pl.ds(sid * rpt, rpt)],
            )

        @pl.when(cid == 1)
        def _():
            pltpu.sync_copy(
                deg_sh.at[pl.ds(sid * rpt, rpt)],
                out1_hbm.at[pl.ds(sid * rpt, rpt)],
            )

    return deg_kernel


def _make_agg_kernel(n_pad, e, d):
    per_tile = e // _NW
    nchunks = per_tile // _K
    tail = per_tile % _K
    rpt = n_pad // _NS
    nsg = nchunks // _NBI
    rem = nchunks % _NBI
    assert nsg >= 2 and tail % 8 == 0

    mesh = plsc.VectorSubcoreMesh(core_axis_name="c", subcore_axis_name="s")

    @functools.partial(
        pl.kernel,
        out_type=jax.ShapeDtypeStruct((_NC, n_pad, d), jnp.float32),
        mesh=mesh,
        scratch_types=[
            pltpu.VMEM((_NBI, _K), jnp.int32),
            pltpu.VMEM((_NBI, _K), jnp.int32),
            pltpu.VMEM((tail if tail else 8,), jnp.int32),
            pltpu.VMEM((tail if tail else 8,), jnp.int32),
            pltpu.VMEM((_NB, _K, d), jnp.float32),
            pltpu.VMEM_SHARED((n_pad, d), jnp.float32),
            pltpu.SemaphoreType.DMA((_NBI,)),
            pltpu.SemaphoreType.DMA((_NB,)),
            pltpu.SemaphoreType.DMA((_NB,)),
        ],
    )
    def agg_kernel(
        y_hbm, edge_hbm, out_hbm,
        srcv, dstv, srcv_t, dstv_t, rows, acc_sh, sem_i, sem_g, sem_s,
    ):
        cid = lax.axis_index("c")
        sid = lax.axis_index("s")
        wid = cid * _NS + sid
        ebase = wid * per_tile

        zero16 = jnp.zeros((_LANES,), jnp.float32)

        def zfill(r, carry):
            for j in range(d // _LANES):
                rows[0, r, pl.ds(j * _LANES, _LANES)] = zero16
            return carry

        lax.fori_loop(0, _K, zfill, 0)
        for t in range(rpt // _K):
            pltpu.sync_copy(rows.at[0], acc_sh.at[pl.ds(sid * rpt + t * _K, _K)])
        plsc.subcore_barrier()

        def issue_idx(i, bI):
            pltpu.async_copy(
                edge_hbm.at[pl.ds(ebase + i * _K, _K)], srcv.at[bI], sem_i.at[bI]
            )
            pltpu.async_copy(
                edge_hbm.at[pl.ds(e + ebase + i * _K, _K)], dstv.at[bI], sem_i.at[bI]
            )

        def wait_idx(i, bI):
            pltpu.make_async_copy(
                edge_hbm.at[pl.ds(ebase + i * _K, _K)], srcv.at[bI], sem_i.at[bI]
            ).wait()
            pltpu.make_async_copy(
                edge_hbm.at[pl.ds(e + ebase + i * _K, _K)], dstv.at[bI], sem_i.at[bI]
            ).wait()

        def issue_gather(bI, bR):
            pltpu.async_copy(y_hbm.at[dstv.at[bI]], rows.at[bR], sem_g.at[bR])

        def wait_gather(bI, bR):
            pltpu.make_async_copy(
                y_hbm.at[dstv.at[bI]], rows.at[bR], sem_g.at[bR]
            ).wait()

        def issue_scat(bI, bR):
            pltpu.async_copy(
                rows.at[bR], acc_sh.at[srcv.at[bI]], sem_s.at[bR], add=True
            )

        def wait_scat(bI, bR):
            pltpu.make_async_copy(
                rows.at[bR], acc_sh.at[srcv.at[bI]], sem_s.at[bR]
            ).wait()

        def do_chunk(i, k, wait_prev_scat, idx_ahead, gather_next):
            wait_gather(k, k % _NB)
            issue_scat(k, k % _NB)
            if wait_prev_scat:
                wait_scat((k - 1) % _NBI, (k - 1) % _NB)
            if idx_ahead:
                issue_idx(i + 2, (k + 2) % _NBI)
            if gather_next:
                wait_idx(i + 1, (k + 1) % _NBI)
                issue_gather((k + 1) % _NBI, (k + 1) % _NB)

        issue_idx(0, 0)
        issue_idx(1, 1)
        wait_idx(0, 0)
        issue_gather(0, 0)

        do_chunk(0, 0, False, True, True)
        for k in range(1, _NBI):
            do_chunk(k, k, True, True, True)

        def sgroup(u, carry):
            i0 = u * _NBI
            for k in range(_NBI):
                do_chunk(i0 + k, k, True, True, True)
            return carry

        if rem == 0:
            lax.fori_loop(1, nsg - 1, sgroup, 0)
            i0 = (nsg - 1) * _NBI
            for k in range(_NBI):
                i = i0 + k
                do_chunk(i, k, True, i + 2 < nchunks, i + 1 < nchunks)
        else:
            lax.fori_loop(1, nsg, sgroup, 0)
            i0 = nsg * _NBI
            for k in range(rem):
                i = i0 + k
                do_chunk(i, k, True, i + 2 < nchunks, i + 1 < nchunks)

        kl = (nchunks - 1) % _NBI
        wait_scat(kl, kl % _NB)

        if tail:
            tb = ebase + nchunks * _K
            pltpu.sync_copy(edge_hbm.at[pl.ds(tb, tail)], srcv_t)
            pltpu.sync_copy(edge_hbm.at[pl.ds(e + tb, tail)], dstv_t)
            pltpu.async_copy(
                y_hbm.at[dstv_t], rows.at[0, pl.ds(0, tail)], sem_g.at[0]
            ).wait()
            pltpu.sync_copy(
                rows.at[0, pl.ds(0, tail)], acc_sh.at[srcv_t], add=True
            )

        plsc.subcore_barrier()
        pltpu.sync_copy(
            acc_sh.at[pl.ds(sid * rpt, rpt)],
            out_hbm.at[cid, pl.ds(sid * rpt, rpt)],
        )

    return agg_kernel


def _prep_body(dpt_ref, x_ref, y_ref):
    deg = dpt_ref[:, 0:1] + dpt_ref[:, 1:2]
    dis = jnp.where(deg > 0.0, lax.rsqrt(deg), 0.0)
    y_ref[...] = x_ref[...] * dis


def _make_fin_body(n):
    def fin_body(acc_ref, out_ref):
        s = acc_ref[0, :n, :] + acc_ref[1, :n, :]
        sq = jnp.sum(s * s, axis=1, keepdims=True)
        t0 = s[:, 0:1]
        li = sq - 2.0 * t0 * t0
        denom = jnp.sqrt(jnp.clip(jnp.abs(li), 1e-8, None))
        out_ref[...] = s * ((1.0 / jnp.sqrt(_C)) / denom)

    return fin_body


def kernel(x, edge_index):
    n, d = x.shape
    e = edge_index.shape[1]
    n_pad = _ceil_to(n, _NS * _K)

    edge_flat = edge_index.reshape(-1)

    dp0, dp1 = _make_deg_kernel(n_pad, e)(edge_flat)
    dpt = jnp.stack([dp0[:n], dp1[:n]], axis=1)

    y = pl.pallas_call(
        _prep_body,
        out_shape=jax.ShapeDtypeStruct((n, d), jnp.float32),
    )(dpt, x)

    acc = _make_agg_kernel(n_pad, e, d)(y, edge_flat)

    out = pl.pallas_call(
        _make_fin_body(n),
        out_shape=jax.ShapeDtypeStruct((n, d), jnp.float32),
    )(acc)
    return out


# speedup vs baseline: 35.8742x; 1.0893x over previous
import functools

import jax
import jax.numpy as jnp
from jax import lax
from jax.experimental import pallas as pl
from jax.experimental.pallas import tpu as pltpu
from jax.experimental.pallas import tpu_sc as plsc

_NC = 2
_NS = 16
_NW = _NC * _NS
_LANES = 16
_K = 128
_NB = 2
_NBI = 2 * _NB
_C = 1.0


def _ceil_to(a, b):
    return (a + b - 1) // b * b


def _make_deg_kernel(n_pad, e):
    per_tile = e // _NW
    nchunks = per_tile // _K
    tail = per_tile % _K
    rpt = n_pad // _NS
    assert nchunks >= 2 * _NBI
    assert tail % 8 == 0 and rpt % _LANES == 0

    mesh = plsc.VectorSubcoreMesh(core_axis_name="c", subcore_axis_name="s")

    @functools.partial(
        pl.kernel,
        out_type=(
            jax.ShapeDtypeStruct((n_pad,), jnp.float32),
            jax.ShapeDtypeStruct((n_pad,), jnp.float32),
        ),
        mesh=mesh,
        scratch_types=[
            pltpu.VMEM((_NBI, _K), jnp.int32),
            pltpu.VMEM((tail if tail else 8,), jnp.int32),
            pltpu.VMEM((_K,), jnp.float32),
            pltpu.VMEM((rpt,), jnp.float32),
            pltpu.VMEM_SHARED((n_pad,), jnp.float32),
            pltpu.SemaphoreType.DMA((_NBI,)),
            pltpu.SemaphoreType.DMA((_NBI,)),
        ],
    )
    def deg_kernel(
        edge_hbm, out0_hbm, out1_hbm, idx_v, idx_t, ones_v, z_v, deg_sh, sem_i, sem_s
    ):
        cid = lax.axis_index("c")
        sid = lax.axis_index("s")
        wid = cid * _NS + sid
        ebase = wid * per_tile

        zero16 = jnp.zeros((_LANES,), jnp.float32)
        one16 = jnp.ones((_LANES,), jnp.float32)

        def zfill(i, carry):
            z_v[pl.ds(i * _LANES, _LANES)] = zero16
            return carry

        lax.fori_loop(0, rpt // _LANES, zfill, 0)
        for j in range(_K // _LANES):
            ones_v[pl.ds(j * _LANES, _LANES)] = one16

        pltpu.sync_copy(z_v, deg_sh.at[pl.ds(sid * rpt, rpt)])
        plsc.subcore_barrier()

        def issue_idx(i, b):
            pltpu.async_copy(
                edge_hbm.at[pl.ds(ebase + i * _K, _K)], idx_v.at[b], sem_i.at[b]
            )

        def wait_idx(i, b):
            pltpu.make_async_copy(
                edge_hbm.at[pl.ds(ebase + i * _K, _K)], idx_v.at[b], sem_i.at[b]
            ).wait()

        def issue_scat(k):
            pltpu.async_copy(ones_v, deg_sh.at[idx_v.at[k]], sem_s.at[k], add=True)

        def wait_scat(k):
            pltpu.make_async_copy(ones_v, deg_sh.at[idx_v.at[k]], sem_s.at[k]).wait()

        def do_chunk(i, k, wait_prev_scat, idx_ahead):
            wait_idx(i, k)
            if wait_prev_scat:
                wait_scat((k + 2) % _NBI)
            if idx_ahead:
                issue_idx(i + 2, (k + 2) % _NBI)
            issue_scat(k)

        nsg = nchunks // _NBI
        rem = nchunks % _NBI

        issue_idx(0, 0)
        issue_idx(1, 1)
        for k in range(_NBI):
            do_chunk(k, k, k >= 2, True)

        def sgroup(u, carry):
            i0 = u * _NBI
            for k in range(_NBI):
                do_chunk(i0 + k, k, True, True)
            return carry

        if rem == 0:
            lax.fori_loop(1, nsg - 1, sgroup, 0)
            i0 = (nsg - 1) * _NBI
            for k in range(_NBI):
                i = i0 + k
                do_chunk(i, k, True, i + 2 < nchunks)
        else:
            lax.fori_loop(1, nsg, sgroup, 0)
            i0 = nsg * _NBI
            for k in range(rem):
                i = i0 + k
                do_chunk(i, k, True, i + 2 < nchunks)

        for i in range(nchunks - 2, nchunks):
            wait_scat(i % _NBI)
        if tail:
            pltpu.sync_copy(edge_hbm.at[pl.ds(ebase + nchunks * _K, tail)], idx_t)
            pltpu.sync_copy(ones_v.at[pl.ds(0, tail)], deg_sh.at[idx_t], add=True)

        plsc.subcore_barrier()

        @pl.when(cid == 0)
        def _():
            pltpu.sync_copy(
                deg_sh.at[pl.ds(sid * rpt, rpt)],
                out0_hbm.at[pl.ds(sid * rpt, rpt)],
            )

        @pl.when(cid == 1)
        def _():
            pltpu.sync_copy(
                deg_sh.at[pl.ds(sid * rpt, rpt)],
                out1_hbm.at[pl.ds(sid * rpt, rpt)],
            )

    return deg_kernel


def _make_agg_kernel(n_pad, e, d):
    per_tile = e // _NW
    nchunks = per_tile // _K
    tail = per_tile % _K
    rpt = n_pad // _NS
    nsg = nchunks // _NBI
    rem = nchunks % _NBI
    assert nsg >= 2 and tail % 8 == 0

    mesh = plsc.VectorSubcoreMesh(core_axis_name="c", subcore_axis_name="s")

    @functools.partial(
        pl.kernel,
        out_type=jax.ShapeDtypeStruct((_NC, n_pad, d), jnp.float32),
        mesh=mesh,
        scratch_types=[
            pltpu.VMEM((_NBI, _K), jnp.int32),
            pltpu.VMEM((_NBI, _K), jnp.int32),
            pltpu.VMEM((tail if tail else 8,), jnp.int32),
            pltpu.VMEM((tail if tail else 8,), jnp.int32),
            pltpu.VMEM((_NB, _K, d), jnp.float32),
            pltpu.VMEM_SHARED((n_pad, d), jnp.float32),
            pltpu.SemaphoreType.DMA((_NBI,)),
            pltpu.SemaphoreType.DMA((_NB,)),
            pltpu.SemaphoreType.DMA((_NB,)),
        ],
    )
    def agg_kernel(
        y_hbm, edge_hbm, out_hbm,
        srcv, dstv, srcv_t, dstv_t, rows, acc_sh, sem_i, sem_g, sem_s,
    ):
        cid = lax.axis_index("c")
        sid = lax.axis_index("s")
        wid = cid * _NS + sid
        ebase = wid * per_tile

        zero16 = jnp.zeros((_LANES,), jnp.float32)

        def zfill(r, carry):
            for j in range(d // _LANES):
                rows[0, r, pl.ds(j * _LANES, _LANES)] = zero16
            return carry

        lax.fori_loop(0, _K, zfill, 0)
        for t in range(rpt // _K):
            pltpu.sync_copy(rows.at[0], acc_sh.at[pl.ds(sid * rpt + t * _K, _K)])
        plsc.subcore_barrier()

        def issue_idx(i, bI):
            pltpu.async_copy(
                edge_hbm.at[pl.ds(ebase + i * _K, _K)], srcv.at[bI], sem_i.at[bI]
            )
            pltpu.async_copy(
                edge_hbm.at[pl.ds(e + ebase + i * _K, _K)], dstv.at[bI], sem_i.at[bI]
            )

        def wait_idx(i, bI):
            pltpu.make_async_copy(
                edge_hbm.at[pl.ds(ebase + i * _K, _K)], srcv.at[bI], sem_i.at[bI]
            ).wait()
            pltpu.make_async_copy(
                edge_hbm.at[pl.ds(e + ebase + i * _K, _K)], dstv.at[bI], sem_i.at[bI]
            ).wait()

        def issue_gather(bI, bR):
            pltpu.async_copy(y_hbm.at[dstv.at[bI]], rows.at[bR], sem_g.at[bR])

        def wait_gather(bI, bR):
            pltpu.make_async_copy(
                y_hbm.at[dstv.at[bI]], rows.at[bR], sem_g.at[bR]
            ).wait()

        def issue_scat(bI, bR):
            pltpu.async_copy(
                rows.at[bR], acc_sh.at[srcv.at[bI]], sem_s.at[bR], add=True
            )

        def wait_scat(bI, bR):
            pltpu.make_async_copy(
                rows.at[bR], acc_sh.at[srcv.at[bI]], sem_s.at[bR]
            ).wait()

        def do_chunk(i, k, wait_prev_scat, idx_ahead, gather_next):
            wait_gather(k, k % _NB)
            issue_scat(k, k % _NB)
            if wait_prev_scat:
                wait_scat((k - 1) % _NBI, (k - 1) % _NB)
            if idx_ahead:
                issue_idx(i + 2, (k + 2) % _NBI)
            if gather_next:
                wait_idx(i + 1, (k + 1) % _NBI)
                issue_gather((k + 1) % _NBI, (k + 1) % _NB)

        issue_idx(0, 0)
        issue_idx(1, 1)
        wait_idx(0, 0)
        issue_gather(0, 0)

        do_chunk(0, 0, False, True, True)
        for k in range(1, _NBI):
            do_chunk(k, k, True, True, True)

        def sgroup(u, carry):
            i0 = u * _NBI
            for k in range(_NBI):
                do_chunk(i0 + k, k, True, True, True)
            return carry

        if rem == 0:
            lax.fori_loop(1, nsg - 1, sgroup, 0)
            i0 = (nsg - 1) * _NBI
            for k in range(_NBI):
                i = i0 + k
                do_chunk(i, k, True, i + 2 < nchunks, i + 1 < nchunks)
        else:
            lax.fori_loop(1, nsg, sgroup, 0)
            i0 = nsg * _NBI
            for k in range(rem):
                i = i0 + k
                do_chunk(i, k, True, i + 2 < nchunks, i + 1 < nchunks)

        kl = (nchunks - 1) % _NBI
        wait_scat(kl, kl % _NB)

        if tail:
            tb = ebase + nchunks * _K
            pltpu.sync_copy(edge_hbm.at[pl.ds(tb, tail)], srcv_t)
            pltpu.sync_copy(edge_hbm.at[pl.ds(e + tb, tail)], dstv_t)
            pltpu.async_copy(
                y_hbm.at[dstv_t], rows.at[0, pl.ds(0, tail)], sem_g.at[0]
            ).wait()
            pltpu.sync_copy(
                rows.at[0, pl.ds(0, tail)], acc_sh.at[srcv_t], add=True
            )

        plsc.subcore_barrier()
        pltpu.sync_copy(
            acc_sh.at[pl.ds(sid * rpt, rpt)],
            out_hbm.at[cid, pl.ds(sid * rpt, rpt)],
        )

    return agg_kernel


def _prep_body(dpt_ref, x_ref, y_ref):
    deg = dpt_ref[:, 0:1] + dpt_ref[:, 1:2]
    dis = jnp.where(deg > 0.0, lax.rsqrt(deg), 0.0)
    y_ref[...] = x_ref[...] * dis


def _make_fin_body(n):
    def fin_body(acc_ref, out_ref):
        s = acc_ref[0, :n, :] + acc_ref[1, :n, :]
        sq = jnp.sum(s * s, axis=1, keepdims=True)
        t0 = s[:, 0:1]
        li = sq - 2.0 * t0 * t0
        denom = jnp.sqrt(jnp.clip(jnp.abs(li), 1e-8, None))
        out_ref[...] = s * ((1.0 / jnp.sqrt(_C)) / denom)

    return fin_body


def kernel(x, edge_index):
    n, d = x.shape
    e = edge_index.shape[1]
    n_pad = _ceil_to(n, _NS * _K)

    edge_flat = edge_index.reshape(-1)

    dp0, dp1 = _make_deg_kernel(n_pad, e)(edge_flat)
    dpt = jnp.stack([dp0[:n], dp1[:n]], axis=1)

    y = pl.pallas_call(
        _prep_body,
        out_shape=jax.ShapeDtypeStruct((n, d), jnp.float32),
    )(dpt, x)

    acc = _make_agg_kernel(n_pad, e, d)(y, edge_flat)

    out = pl.pallas_call(
        _make_fin_body(n),
        out_shape=jax.ShapeDtypeStruct((n, d), jnp.float32),
    )(acc)
    return out
